# baseline (speedup 1.0000x reference)
import functools

import jax
import jax.numpy as jnp
from jax import lax
from jax.experimental import pallas as pl
from jax.experimental.pallas import tpu as pltpu
from jax.experimental.pallas import tpu_sc as plsc


@functools.lru_cache(maxsize=None)
def _make_gather(B, DU, DM):
    info = plsc.get_sparse_core_info()
    NC, NS = info.num_cores, info.num_subcores
    NW = NC * NS
    b_per_w = B // NW
    CH = 128
    n_ch = b_per_w // CH
    mesh = plsc.VectorSubcoreMesh(core_axis_name="c", subcore_axis_name="s")

    @functools.partial(
        pl.kernel,
        mesh=mesh,
        out_type=(jax.ShapeDtypeStruct((B, DU), jnp.float32),
                  jax.ShapeDtypeStruct((B, DM), jnp.float32)),
        scratch_types=[
            pltpu.VMEM((n_ch, CH), jnp.int32),
            pltpu.VMEM((n_ch, CH), jnp.int32),
            pltpu.VMEM((b_per_w, DU), jnp.float32),
            pltpu.VMEM((b_per_w, DM), jnp.float32),
            pltpu.SemaphoreType.DMA,
        ],
        compiler_params=pltpu.CompilerParams(use_tc_tiling_on_sc=False),
    )
    def gather(uid_hbm, mid_hbm, utab_hbm, mtab_hbm, out_u, out_m,
               uidx_v, midx_v, urows_v, mrows_v, sem):
        wid = lax.axis_index("s") * NC + lax.axis_index("c")
        base = wid * b_per_w
        pltpu.sync_copy(uid_hbm.at[wid], uidx_v)
        pltpu.sync_copy(mid_hbm.at[wid], midx_v)
        copies = []
        for c in range(n_ch):
            copies.append(pltpu.async_copy(
                utab_hbm.at[uidx_v.at[c]], urows_v.at[pl.ds(c * CH, CH)], sem))
            copies.append(pltpu.async_copy(
                mtab_hbm.at[midx_v.at[c]], mrows_v.at[pl.ds(c * CH, CH)], sem))
        for cp in copies:
            cp.wait()
        pltpu.sync_copy(urows_v, out_u.at[pl.ds(base, b_per_w)])
        pltpu.sync_copy(mrows_v, out_m.at[pl.ds(base, b_per_w)])

    return gather, NW, n_ch, CH


def _mlp_body(ue_ref, me_ref, Wu_ref, bu_ref, Wm_ref, bm_ref,
              W1u_ref, W1m_ref, b1_ref, W2_ref, b2_ref, W3_ref, b3_ref,
              out_ref):
    dn = (((1,), (1,)), ((), ()))
    f32 = jnp.float32
    u = lax.dot_general(ue_ref[...], Wu_ref[...], dn,
                        preferred_element_type=f32) + bu_ref[...]
    m = lax.dot_general(me_ref[...], Wm_ref[...], dn,
                        preferred_element_type=f32) + bm_ref[...]
    x1 = (lax.dot_general(u, W1u_ref[...], dn, preferred_element_type=f32)
          + lax.dot_general(m, W1m_ref[...], dn, preferred_element_type=f32)
          + b1_ref[...])
    h = lax.dot_general(jnp.maximum(x1, 0.0), W2_ref[...], dn,
                        preferred_element_type=f32) + b2_ref[...] + x1
    o = jnp.sum(jnp.maximum(h, 0.0) * W3_ref[...], axis=1, keepdims=True)
    out_ref[...] = jax.nn.sigmoid(o + b3_ref[...])


@functools.lru_cache(maxsize=None)
def _make_mlp(B, DU, DM, H):
    BLK = 2048
    grid = (B // BLK,)
    full = lambda shape: pl.BlockSpec(shape, lambda i: (0,) * len(shape))
    return pl.pallas_call(
        _mlp_body,
        grid=grid,
        in_specs=[
            pl.BlockSpec((BLK, DU), lambda i: (i, 0)),
            pl.BlockSpec((BLK, DM), lambda i: (i, 0)),
            full((DU, DU)), full((1, DU)),
            full((DM, DM)), full((1, DM)),
            full((H, DU)), full((H, DM)), full((1, H)),
            full((H, H)), full((1, H)),
            full((1, H)), full((1, 1)),
        ],
        out_specs=pl.BlockSpec((BLK, 1), lambda i: (i, 0)),
        out_shape=jax.ShapeDtypeStruct((B, 1), jnp.float32),
        compiler_params=pltpu.CompilerParams(
            dimension_semantics=("arbitrary",),
        ),
    )


def kernel(user_id, movie_id, user_table, movie_table,
           W_u, b_u, W_m, b_m, W1, b1, W2, b2, W3, b3):
    B = user_id.shape[0]
    DU = user_table.shape[1]
    DM = movie_table.shape[1]
    H = W1.shape[0]

    gather, NW, n_ch, CH = _make_gather(B, DU, DM)
    uid = user_id.astype(jnp.int32).reshape(NW, n_ch, CH)
    mid = movie_id.astype(jnp.int32).reshape(NW, n_ch, CH)
    ue, me = gather(uid, mid, user_table, movie_table)

    mlp = _make_mlp(B, DU, DM, H)
    return mlp(ue, me,
               W_u, b_u.reshape(1, DU),
               W_m, b_m.reshape(1, DM),
               W1[:, :DU], W1[:, DU:], b1.reshape(1, H),
               W2, b2.reshape(1, H),
               W3, b3.reshape(1, 1))
